# baseline (speedup 1.0000x reference)
import jax
import jax.numpy as jnp
from jax import lax
from jax.experimental import pallas as pl
from jax.experimental.pallas import tpu as pltpu
from jax.experimental.pallas import tpu_sc as plsc

N_USERS = 20000
N_NEWS = 10000
N_ENT = 25000
N_NODE = N_NEWS + N_ENT
D = 100
DP = 104
WLO = 56
WHI = 48
NEIGH = 20
NNZ = 500000

NC = 2
NS = 16
G = 128
NBUF = 5
KPRE = 2

NSLAB = 8
ROWS_P = N_NODE // NSLAB
ACC_A = 4480
SA = ACC_A // NS
GPA = 45
EPS_A = NS * GPA * G
PAD_A = EPS_A - ROWS_P * NEIGH

ACC_B = 20096
SB = ACC_B // NS
GPB = 125
CHB = (45, 45, 35)
EPS_B = NS * GPB * G
PAD_B = EPS_B - NNZ // 2


def _pipe_loop(table_hbm, acc, gslot, sslot, bufs, sgs, sss, n_groups):
    for b in range(KPRE):
        pltpu.async_copy(table_hbm.at[gslot.at[b]], bufs[b], sgs[b])

    def body(i, carry):
        g0 = NBUF * i
        for db in range(NBUF):
            g = g0 + db
            bn = (db + KPRE) % NBUF
            pltpu.make_async_copy(
                table_hbm.at[gslot.at[g]], bufs[db], sgs[db]).wait()
            if db < NBUF - KPRE:
                @pl.when(g0 > 0)
                def _():
                    pltpu.make_async_copy(
                        bufs[bn], acc.at[sslot.at[0]], sss[bn]).wait()
            else:
                pltpu.make_async_copy(
                    bufs[bn], acc.at[sslot.at[0]], sss[bn]).wait()

            @pl.when(g + KPRE < n_groups)
            def _():
                pltpu.async_copy(table_hbm.at[gslot.at[g + KPRE]],
                                 bufs[bn], sgs[bn])

            pltpu.async_copy(bufs[db], acc.at[sslot.at[g]], sss[db], add=True)
        return carry

    lax.fori_loop(0, n_groups // NBUF, body, 0)
    for b in range(KPRE, NBUF):
        pltpu.make_async_copy(bufs[b], acc.at[sslot.at[0]], sss[b]).wait()


def _agg_body(gidx_hbm, sidx_hbm, table_hbm, init_hbm, out_hbm,
              acc, gslot, sslot, *bufs_sems):
    bufs, sgs, sss = (bufs_sems[0:NBUF], bufs_sems[NBUF:2 * NBUF],
                      bufs_sems[2 * NBUF:3 * NBUF])
    sc = lax.axis_index("c")
    t = lax.axis_index("s")
    for p in range(NSLAB // NC):
        slab = sc * (NSLAB // NC) + p
        pltpu.sync_copy(init_hbm.at[pl.ds(slab * ACC_A + t * SA, SA)],
                        acc.at[pl.ds(t * SA, SA)])
        plsc.subcore_barrier()
        row0 = (slab * NS + t) * GPA
        pltpu.sync_copy(gidx_hbm.at[pl.ds(row0, GPA)], gslot)
        pltpu.sync_copy(sidx_hbm.at[pl.ds(row0, GPA)], sslot)
        _pipe_loop(table_hbm, acc, gslot, sslot, bufs, sgs, sss, GPA)
        plsc.subcore_barrier()
        pltpu.sync_copy(acc.at[pl.ds(t * SA, SA)],
                        out_hbm.at[slab, pl.ds(t * SA, SA)])


def _seg_body(gidx_hbm, sidx_hbm, table_hbm, init_hbm, out_hbm,
              acc, gslot, sslot, *bufs_sems):
    bufs, sgs, sss = (bufs_sems[0:NBUF], bufs_sems[NBUF:2 * NBUF],
                      bufs_sems[2 * NBUF:3 * NBUF])
    sc = lax.axis_index("c")
    t = lax.axis_index("s")
    pltpu.sync_copy(init_hbm.at[pl.ds(t * SB, SB)], acc.at[pl.ds(t * SB, SB)])
    plsc.subcore_barrier()
    base = (sc * NS + t) * GPB
    off = 0
    for ch in CHB:
        pltpu.sync_copy(gidx_hbm.at[pl.ds(base + off, ch)], gslot.at[pl.ds(0, ch)])
        pltpu.sync_copy(sidx_hbm.at[pl.ds(base + off, ch)], sslot.at[pl.ds(0, ch)])
        _pipe_loop(table_hbm, acc, gslot, sslot, bufs, sgs, sss, ch)
        off += ch
    plsc.subcore_barrier()
    pltpu.sync_copy(acc.at[pl.ds(t * SB, SB)],
                    out_hbm.at[sc, pl.ds(t * SB, SB)])


def _sc_call(body, n_out_major, acc_rows, n_groups, width):
    return pl.kernel(
        body,
        out_type=jax.ShapeDtypeStruct((n_out_major, acc_rows, width),
                                      jnp.float32),
        mesh=plsc.VectorSubcoreMesh(core_axis_name="c", subcore_axis_name="s",
                                    num_cores=NC, num_subcores=NS),
        scratch_types=(
            [pltpu.VMEM_SHARED((acc_rows, width), jnp.float32),
             pltpu.VMEM((n_groups, G), jnp.int32),
             pltpu.VMEM((n_groups, G), jnp.int32)]
            + [pltpu.VMEM((G, width), jnp.float32)] * NBUF
            + [pltpu.SemaphoreType.DMA] * (2 * NBUF)
        ),
        compiler_params=pltpu.CompilerParams(use_tc_tiling_on_sc=False),
    )


def _norm_body(x_ref, a_ref, o_ref):
    x = x_ref[...]
    n = jnp.maximum(jnp.sqrt(jnp.sum(x * x, axis=1, keepdims=True)), 1e-12)
    o_ref[...] = a_ref[...] + 2.0 * (x[:, :D] / n)


def _user_body(u_ref, plo_ref, phi_ref, o_ref):
    u = u_ref[...]
    agg = jnp.concatenate(
        [plo_ref[0] + plo_ref[1], (phi_ref[0] + phi_ref[1])[:, :D - WLO]],
        axis=1)
    x = u + agg
    n = jnp.maximum(jnp.sqrt(jnp.sum(x * x, axis=1, keepdims=True)), 1e-12)
    o_ref[...] = u + 2.0 * (x / n)


def kernel(user_embedding, all_embedding, entity_embedding, relation_embedding,
           news_entities, neigh_entities, neigh_relations,
           interact_rows, interact_cols, interact_vals,
           W_news, b_news, W_ent, b_ent):
    f32, i32 = jnp.float32, jnp.int32

    ent_pad = jnp.pad(entity_embedding, ((0, 0), (0, DP - D)))
    ent_idx = jnp.concatenate(
        [news_entities.reshape(-1), neigh_entities.reshape(-1)])
    zpa = jnp.zeros((PAD_A,), i32)
    gidx_a = jnp.concatenate(
        [jnp.concatenate([ent_idx[s * ROWS_P * NEIGH:(s + 1) * ROWS_P * NEIGH],
                          zpa]) for s in range(NSLAB)]).reshape(-1, G)
    sidx_slab = jnp.concatenate(
        [jnp.repeat(jnp.arange(ROWS_P, dtype=i32), NEIGH),
         jnp.full((PAD_A,), ROWS_P, i32)])
    sidx_a = jnp.tile(sidx_slab, NSLAB).reshape(-1, G)
    base = jnp.pad(
        jnp.concatenate([all_embedding[:N_NEWS], all_embedding[:N_ENT]],
                        axis=0),
        ((0, 0), (0, DP - D)))
    zrows = jnp.zeros((ACC_A - ROWS_P, DP), f32)
    init_a = jnp.concatenate(
        [jnp.concatenate([base[s * ROWS_P:(s + 1) * ROWS_P], zrows])
         for s in range(NSLAB)])

    parts_a = _sc_call(_agg_body, NSLAB, ACC_A, GPA, DP)(
        gidx_a, sidx_a, ent_pad, init_a)
    node_raw = jnp.concatenate(
        [parts_a[s, :ROWS_P] for s in range(NSLAB)], axis=0)

    h = NNZ // 2
    zpb = jnp.zeros((PAD_B,), i32)
    upb = jnp.full((PAD_B,), N_USERS, i32)
    gidx_b = jnp.concatenate(
        [interact_cols[:h], zpb, interact_cols[h:], zpb]).reshape(-1, G)
    sidx_b = jnp.concatenate(
        [interact_rows[:h], upb, interact_rows[h:], upb]).reshape(-1, G)

    parts_lo = _sc_call(_seg_body, NC, ACC_B, max(CHB), WLO)(
        gidx_b, sidx_b, node_raw[:, :WLO], jnp.zeros((ACC_B, WLO), f32))
    parts_hi = _sc_call(_seg_body, NC, ACC_B, max(CHB), WHI)(
        gidx_b, sidx_b, node_raw[:, WLO:], jnp.zeros((ACC_B, WHI), f32))

    bl = 1000
    node_res = pl.pallas_call(
        _norm_body,
        out_shape=jax.ShapeDtypeStruct((N_NODE, D), f32),
        grid=(N_NODE // bl,),
        in_specs=[pl.BlockSpec((bl, DP), lambda i: (i, 0)),
                  pl.BlockSpec((bl, D), lambda i: (i, 0))],
        out_specs=pl.BlockSpec((bl, D), lambda i: (i, 0)),
    )(node_raw, all_embedding)

    user_res = pl.pallas_call(
        _user_body,
        out_shape=jax.ShapeDtypeStruct((N_USERS, D), f32),
        grid=(N_USERS // bl,),
        in_specs=[pl.BlockSpec((bl, D), lambda i: (i, 0)),
                  pl.BlockSpec((NC, bl, WLO), lambda i: (0, i, 0)),
                  pl.BlockSpec((NC, bl, WHI), lambda i: (0, i, 0))],
        out_specs=pl.BlockSpec((bl, D), lambda i: (i, 0)),
    )(user_embedding, parts_lo, parts_hi)

    return (user_res, node_res)


# speedup vs baseline: 1.3614x; 1.3614x over previous
import jax
import jax.numpy as jnp
from jax import lax
from jax.experimental import pallas as pl
from jax.experimental.pallas import tpu as pltpu
from jax.experimental.pallas import tpu_sc as plsc

N_USERS = 20000
N_NEWS = 10000
N_ENT = 25000
N_NODE = N_NEWS + N_ENT
D = 100
DP = 104
WLO = 56
WHI = 48
NEIGH = 20
NNZ = 500000

NC = 2
NS = 16
G = 256

NSLAB = 4
ROWS_P = N_NODE // NSLAB
ACC_A = 8832
SA = ACC_A // NS
GPA = 44
CHA = (22, 22)
EPS_A = NS * GPA * G
PAD_A = EPS_A - ROWS_P * NEIGH

ACC_B = 20096
SB = ACC_B // NS
GPB = 62
CHB = (32, 30)
EPS_B = NS * GPB * G
PAD_B = EPS_B - NNZ // 2


def _pipe_loop(table_hbm, acc, gslot, sslot, bufs, sgs, sss, n_groups):
    pltpu.async_copy(table_hbm.at[gslot.at[0]], bufs[0], sgs[0])

    def body(i, carry):
        g0 = 2 * i
        pltpu.make_async_copy(table_hbm.at[gslot.at[g0]], bufs[0], sgs[0]).wait()

        @pl.when(g0 > 0)
        def _():
            pltpu.make_async_copy(bufs[1], acc.at[sslot.at[0]], sss[1]).wait()

        pltpu.async_copy(table_hbm.at[gslot.at[g0 + 1]], bufs[1], sgs[1])
        pltpu.async_copy(bufs[0], acc.at[sslot.at[g0]], sss[0], add=True)
        pltpu.make_async_copy(
            table_hbm.at[gslot.at[g0 + 1]], bufs[1], sgs[1]).wait()
        pltpu.make_async_copy(bufs[0], acc.at[sslot.at[0]], sss[0]).wait()

        @pl.when(g0 + 2 < n_groups)
        def _():
            pltpu.async_copy(table_hbm.at[gslot.at[g0 + 2]], bufs[0], sgs[0])

        pltpu.async_copy(bufs[1], acc.at[sslot.at[g0 + 1]], sss[1], add=True)
        return carry

    lax.fori_loop(0, n_groups // 2, body, 0)
    pltpu.make_async_copy(bufs[1], acc.at[sslot.at[0]], sss[1]).wait()


def _run_chunks(gidx_hbm, sidx_hbm, table_hbm, acc, gslot, sslot,
                bufs, sgs, sss, base, chunks):
    off = 0
    for ch in chunks:
        pltpu.sync_copy(gidx_hbm.at[pl.ds(base + off, ch)],
                        gslot.at[pl.ds(0, ch)])
        pltpu.sync_copy(sidx_hbm.at[pl.ds(base + off, ch)],
                        sslot.at[pl.ds(0, ch)])
        _pipe_loop(table_hbm, acc, gslot, sslot, bufs, sgs, sss, ch)
        off += ch


def _agg_body(gidx_hbm, sidx_hbm, table_hbm, init_hbm, out_hbm,
              acc, gslot, sslot, buf0, buf1, sg0, sg1, ss0, ss1):
    sc = lax.axis_index("c")
    t = lax.axis_index("s")
    for p in range(NSLAB // NC):
        slab = sc * (NSLAB // NC) + p
        pltpu.sync_copy(init_hbm.at[pl.ds(slab * ACC_A + t * SA, SA)],
                        acc.at[pl.ds(t * SA, SA)])
        plsc.subcore_barrier()
        _run_chunks(gidx_hbm, sidx_hbm, table_hbm, acc, gslot, sslot,
                    (buf0, buf1), (sg0, sg1), (ss0, ss1),
                    (slab * NS + t) * GPA, CHA)
        plsc.subcore_barrier()
        pltpu.sync_copy(acc.at[pl.ds(t * SA, SA)],
                        out_hbm.at[slab, pl.ds(t * SA, SA)])


def _seg_body(gidx_hbm, sidx_hbm, table_hbm, init_hbm, out_hbm,
              acc, gslot, sslot, buf0, buf1, sg0, sg1, ss0, ss1):
    sc = lax.axis_index("c")
    t = lax.axis_index("s")
    pltpu.sync_copy(init_hbm.at[pl.ds(t * SB, SB)], acc.at[pl.ds(t * SB, SB)])
    plsc.subcore_barrier()
    _run_chunks(gidx_hbm, sidx_hbm, table_hbm, acc, gslot, sslot,
                (buf0, buf1), (sg0, sg1), (ss0, ss1),
                (sc * NS + t) * GPB, CHB)
    plsc.subcore_barrier()
    pltpu.sync_copy(acc.at[pl.ds(t * SB, SB)],
                    out_hbm.at[sc, pl.ds(t * SB, SB)])


def _sc_call(body, n_out_major, acc_rows, slot_rows, width):
    return pl.kernel(
        body,
        out_type=jax.ShapeDtypeStruct((n_out_major, acc_rows, width),
                                      jnp.float32),
        mesh=plsc.VectorSubcoreMesh(core_axis_name="c", subcore_axis_name="s",
                                    num_cores=NC, num_subcores=NS),
        scratch_types=[
            pltpu.VMEM_SHARED((acc_rows, width), jnp.float32),
            pltpu.VMEM((slot_rows, G), jnp.int32),
            pltpu.VMEM((slot_rows, G), jnp.int32),
            pltpu.VMEM((G, width), jnp.float32),
            pltpu.VMEM((G, width), jnp.float32),
            pltpu.SemaphoreType.DMA,
            pltpu.SemaphoreType.DMA,
            pltpu.SemaphoreType.DMA,
            pltpu.SemaphoreType.DMA,
        ],
        compiler_params=pltpu.CompilerParams(use_tc_tiling_on_sc=False),
    )


def _norm_body(x_ref, a_ref, o_ref):
    x = x_ref[...]
    n = jnp.maximum(jnp.sqrt(jnp.sum(x * x, axis=1, keepdims=True)), 1e-12)
    o_ref[...] = a_ref[...] + 2.0 * (x[:, :D] / n)


def _user_body(u_ref, plo_ref, phi_ref, o_ref):
    u = u_ref[...]
    agg = jnp.concatenate(
        [plo_ref[0] + plo_ref[1], (phi_ref[0] + phi_ref[1])[:, :D - WLO]],
        axis=1)
    x = u + agg
    n = jnp.maximum(jnp.sqrt(jnp.sum(x * x, axis=1, keepdims=True)), 1e-12)
    o_ref[...] = u + 2.0 * (x / n)


def kernel(user_embedding, all_embedding, entity_embedding, relation_embedding,
           news_entities, neigh_entities, neigh_relations,
           interact_rows, interact_cols, interact_vals,
           W_news, b_news, W_ent, b_ent):
    f32, i32 = jnp.float32, jnp.int32

    ent_pad = jnp.pad(entity_embedding, ((0, 0), (0, DP - D)))
    ent_idx = jnp.concatenate(
        [news_entities.reshape(-1), neigh_entities.reshape(-1)])
    zpa = jnp.zeros((PAD_A,), i32)
    gidx_a = jnp.concatenate(
        [jnp.concatenate([ent_idx[s * ROWS_P * NEIGH:(s + 1) * ROWS_P * NEIGH],
                          zpa]) for s in range(NSLAB)]).reshape(-1, G)
    sidx_slab = jnp.concatenate(
        [jnp.repeat(jnp.arange(ROWS_P, dtype=i32), NEIGH),
         jnp.full((PAD_A,), ROWS_P, i32)])
    sidx_a = jnp.tile(sidx_slab, NSLAB).reshape(-1, G)
    base = jnp.pad(
        jnp.concatenate([all_embedding[:N_NEWS], all_embedding[:N_ENT]],
                        axis=0),
        ((0, 0), (0, DP - D)))
    zrows = jnp.zeros((ACC_A - ROWS_P, DP), f32)
    init_a = jnp.concatenate(
        [jnp.concatenate([base[s * ROWS_P:(s + 1) * ROWS_P], zrows])
         for s in range(NSLAB)])

    parts_a = _sc_call(_agg_body, NSLAB, ACC_A, max(CHA), DP)(
        gidx_a, sidx_a, ent_pad, init_a)
    node_raw = jnp.concatenate(
        [parts_a[s, :ROWS_P] for s in range(NSLAB)], axis=0)

    h = NNZ // 2
    zpb = jnp.zeros((PAD_B,), i32)
    upb = jnp.full((PAD_B,), N_USERS, i32)
    gidx_b = jnp.concatenate(
        [interact_cols[:h], zpb, interact_cols[h:], zpb]).reshape(-1, G)
    sidx_b = jnp.concatenate(
        [interact_rows[:h], upb, interact_rows[h:], upb]).reshape(-1, G)

    parts_lo = _sc_call(_seg_body, NC, ACC_B, max(CHB), WLO)(
        gidx_b, sidx_b, node_raw[:, :WLO], jnp.zeros((ACC_B, WLO), f32))
    parts_hi = _sc_call(_seg_body, NC, ACC_B, max(CHB), WHI)(
        gidx_b, sidx_b, node_raw[:, WLO:], jnp.zeros((ACC_B, WHI), f32))

    bl = 1000
    node_res = pl.pallas_call(
        _norm_body,
        out_shape=jax.ShapeDtypeStruct((N_NODE, D), f32),
        grid=(N_NODE // bl,),
        in_specs=[pl.BlockSpec((bl, DP), lambda i: (i, 0)),
                  pl.BlockSpec((bl, D), lambda i: (i, 0))],
        out_specs=pl.BlockSpec((bl, D), lambda i: (i, 0)),
    )(node_raw, all_embedding)

    user_res = pl.pallas_call(
        _user_body,
        out_shape=jax.ShapeDtypeStruct((N_USERS, D), f32),
        grid=(N_USERS // bl,),
        in_specs=[pl.BlockSpec((bl, D), lambda i: (i, 0)),
                  pl.BlockSpec((NC, bl, WLO), lambda i: (0, i, 0)),
                  pl.BlockSpec((NC, bl, WHI), lambda i: (0, i, 0))],
        out_specs=pl.BlockSpec((bl, D), lambda i: (i, 0)),
    )(user_embedding, parts_lo, parts_hi)

    return (user_res, node_res)


# speedup vs baseline: 1.9757x; 1.4512x over previous
import jax
import jax.numpy as jnp
from jax import lax
from jax.experimental import pallas as pl
from jax.experimental.pallas import tpu as pltpu
from jax.experimental.pallas import tpu_sc as plsc

N_USERS = 20000
N_NEWS = 10000
N_ENT = 25000
N_NODE = N_NEWS + N_ENT
D = 100
DP = 104
WLO = 56
WHI = 48
NEIGH = 20
NNZ = 500000

NC = 2
NS = 16
NW = NC * NS
OFFS = (0, 16, 32, 48, 64, 80, 88)

RPG = 12
TPR = 1104
GPA = TPR // RPG
N_PAD = NW * TPR

G = 256
ACC_B = 20096
SB = ACC_B // NS
GPB = 62
CHB = (32, 30)
EPS_B = NS * GPB * G
PAD_B = EPS_B - NNZ // 2


def _agg_body(gidx_hbm, base_hbm, table_hbm, lo_hbm, hi_hbm,
              islot, gb0, gb1, bb0, bb1, ob0, ob1,
              sg0, sg1, sb0, sb1, sl0, sl1, sh0, sh1):
    gbufs, bbufs, obufs = (gb0, gb1), (bb0, bb1), (ob0, ob1)
    sgs, sbs, sls, shs = (sg0, sg1), (sb0, sb1), (sl0, sl1), (sh0, sh1)
    tg = lax.axis_index("c") * NS + lax.axis_index("s")
    row_t = tg * TPR
    pltpu.sync_copy(gidx_hbm.at[pl.ds(tg * GPA, GPA)], islot)
    for s in range(2):
        pltpu.async_copy(table_hbm.at[islot.at[s]], gbufs[s], sgs[s])
        pltpu.async_copy(base_hbm.at[pl.ds(row_t + s * RPG, RPG)],
                         bbufs[s], sbs[s])

    def body(i, carry):
        for s in range(2):
            g = 2 * i + s
            rows = row_t + g * RPG
            pltpu.make_async_copy(
                table_hbm.at[islot.at[g]], gbufs[s], sgs[s]).wait()
            pltpu.make_async_copy(
                base_hbm.at[pl.ds(rows, RPG)], bbufs[s], sbs[s]).wait()

            @pl.when(i > 0)
            def _():
                pltpu.make_async_copy(
                    obufs[s].at[:, pl.ds(0, WLO)],
                    lo_hbm.at[pl.ds(rows, RPG)], sls[s]).wait()
                pltpu.make_async_copy(
                    obufs[s].at[:, pl.ds(WLO, WHI)],
                    hi_hbm.at[pl.ds(rows, RPG)], shs[s]).wait()

            def red(r, c):
                for off in OFFS:
                    acc = bbufs[s][r, pl.ds(off, 16)]
                    for j in range(NEIGH):
                        acc = acc + gbufs[s][r * NEIGH + j, pl.ds(off, 16)]
                    obufs[s][r, pl.ds(off, 16)] = acc
                return c

            lax.fori_loop(0, RPG, red, 0)
            pltpu.async_copy(obufs[s].at[:, pl.ds(0, WLO)],
                             lo_hbm.at[pl.ds(rows, RPG)], sls[s])
            pltpu.async_copy(obufs[s].at[:, pl.ds(WLO, WHI)],
                             hi_hbm.at[pl.ds(rows, RPG)], shs[s])

            @pl.when(i < GPA // 2 - 1)
            def _():
                pltpu.async_copy(table_hbm.at[islot.at[g + 2]],
                                 gbufs[s], sgs[s])
                pltpu.async_copy(base_hbm.at[pl.ds(rows + 2 * RPG, RPG)],
                                 bbufs[s], sbs[s])
        return carry

    lax.fori_loop(0, GPA // 2, body, 0)
    for s in range(2):
        rows = row_t + (GPA - 2 + s) * RPG
        pltpu.make_async_copy(obufs[s].at[:, pl.ds(0, WLO)],
                              lo_hbm.at[pl.ds(rows, RPG)], sls[s]).wait()
        pltpu.make_async_copy(obufs[s].at[:, pl.ds(WLO, WHI)],
                              hi_hbm.at[pl.ds(rows, RPG)], shs[s]).wait()


def _agg_call():
    return pl.kernel(
        _agg_body,
        out_type=(jax.ShapeDtypeStruct((N_PAD, WLO), jnp.float32),
                  jax.ShapeDtypeStruct((N_PAD, WHI), jnp.float32)),
        mesh=plsc.VectorSubcoreMesh(core_axis_name="c", subcore_axis_name="s",
                                    num_cores=NC, num_subcores=NS),
        scratch_types=(
            [pltpu.VMEM((GPA, RPG * NEIGH), jnp.int32)]
            + [pltpu.VMEM((RPG * NEIGH, DP), jnp.float32)] * 2
            + [pltpu.VMEM((RPG, DP), jnp.float32)] * 4
            + [pltpu.SemaphoreType.DMA] * 8
        ),
        compiler_params=pltpu.CompilerParams(use_tc_tiling_on_sc=False),
    )


def _pipe_loop(table_hbm, acc, gslot, sslot, bufs, sgs, sss, n_groups):
    pltpu.async_copy(table_hbm.at[gslot.at[0]], bufs[0], sgs[0])

    def body(i, carry):
        g0 = 2 * i
        pltpu.make_async_copy(table_hbm.at[gslot.at[g0]], bufs[0], sgs[0]).wait()

        @pl.when(g0 > 0)
        def _():
            pltpu.make_async_copy(bufs[1], acc.at[sslot.at[0]], sss[1]).wait()

        pltpu.async_copy(table_hbm.at[gslot.at[g0 + 1]], bufs[1], sgs[1])
        pltpu.async_copy(bufs[0], acc.at[sslot.at[g0]], sss[0], add=True)
        pltpu.make_async_copy(
            table_hbm.at[gslot.at[g0 + 1]], bufs[1], sgs[1]).wait()
        pltpu.make_async_copy(bufs[0], acc.at[sslot.at[0]], sss[0]).wait()

        @pl.when(g0 + 2 < n_groups)
        def _():
            pltpu.async_copy(table_hbm.at[gslot.at[g0 + 2]], bufs[0], sgs[0])

        pltpu.async_copy(bufs[1], acc.at[sslot.at[g0 + 1]], sss[1], add=True)
        return carry

    lax.fori_loop(0, n_groups // 2, body, 0)
    pltpu.make_async_copy(bufs[1], acc.at[sslot.at[0]], sss[1]).wait()


def _seg_body(gidx_hbm, sidx_hbm, table_hbm, init_hbm, out_hbm,
              acc, gslot, sslot, buf0, buf1, sg0, sg1, ss0, ss1):
    sc = lax.axis_index("c")
    t = lax.axis_index("s")
    pltpu.sync_copy(init_hbm.at[pl.ds(t * SB, SB)], acc.at[pl.ds(t * SB, SB)])
    plsc.subcore_barrier()
    base = (sc * NS + t) * GPB
    off = 0
    for ch in CHB:
        pltpu.sync_copy(gidx_hbm.at[pl.ds(base + off, ch)],
                        gslot.at[pl.ds(0, ch)])
        pltpu.sync_copy(sidx_hbm.at[pl.ds(base + off, ch)],
                        sslot.at[pl.ds(0, ch)])
        _pipe_loop(table_hbm, acc, gslot, sslot, (buf0, buf1),
                   (sg0, sg1), (ss0, ss1), ch)
        off += ch
    plsc.subcore_barrier()
    pltpu.sync_copy(acc.at[pl.ds(t * SB, SB)],
                    out_hbm.at[sc, pl.ds(t * SB, SB)])


def _seg_call(width):
    return pl.kernel(
        _seg_body,
        out_type=jax.ShapeDtypeStruct((NC, ACC_B, width), jnp.float32),
        mesh=plsc.VectorSubcoreMesh(core_axis_name="c", subcore_axis_name="s",
                                    num_cores=NC, num_subcores=NS),
        scratch_types=[
            pltpu.VMEM_SHARED((ACC_B, width), jnp.float32),
            pltpu.VMEM((max(CHB), G), jnp.int32),
            pltpu.VMEM((max(CHB), G), jnp.int32),
            pltpu.VMEM((G, width), jnp.float32),
            pltpu.VMEM((G, width), jnp.float32),
            pltpu.SemaphoreType.DMA,
            pltpu.SemaphoreType.DMA,
            pltpu.SemaphoreType.DMA,
            pltpu.SemaphoreType.DMA,
        ],
        compiler_params=pltpu.CompilerParams(use_tc_tiling_on_sc=False),
    )


def _norm_body(lo_ref, hi_ref, a_ref, o_ref):
    x = jnp.concatenate([lo_ref[...], hi_ref[...][:, :D - WLO]], axis=1)
    n = jnp.maximum(jnp.sqrt(jnp.sum(x * x, axis=1, keepdims=True)), 1e-12)
    o_ref[...] = a_ref[...] + 2.0 * (x / n)


def _user_body(u_ref, plo_ref, phi_ref, o_ref):
    u = u_ref[...]
    agg = jnp.concatenate(
        [plo_ref[0] + plo_ref[1], (phi_ref[0] + phi_ref[1])[:, :D - WLO]],
        axis=1)
    x = u + agg
    n = jnp.maximum(jnp.sqrt(jnp.sum(x * x, axis=1, keepdims=True)), 1e-12)
    o_ref[...] = u + 2.0 * (x / n)


def kernel(user_embedding, all_embedding, entity_embedding, relation_embedding,
           news_entities, neigh_entities, neigh_relations,
           interact_rows, interact_cols, interact_vals,
           W_news, b_news, W_ent, b_ent):
    f32, i32 = jnp.float32, jnp.int32

    ent_pad = jnp.pad(entity_embedding, ((0, 0), (0, DP - D)))
    gidx_a = jnp.concatenate(
        [news_entities.reshape(-1), neigh_entities.reshape(-1),
         jnp.zeros(((N_PAD - N_NODE) * NEIGH,), i32)]).reshape(-1, RPG * NEIGH)
    base_pad = jnp.pad(
        jnp.concatenate([all_embedding[:N_NEWS], all_embedding[:N_ENT]],
                        axis=0),
        ((0, N_PAD - N_NODE), (0, DP - D)))

    node_lo, node_hi = _agg_call()(gidx_a, base_pad, ent_pad)

    h = NNZ // 2
    zpb = jnp.zeros((PAD_B,), i32)
    upb = jnp.full((PAD_B,), N_USERS, i32)
    gidx_b = jnp.concatenate(
        [interact_cols[:h], zpb, interact_cols[h:], zpb]).reshape(-1, G)
    sidx_b = jnp.concatenate(
        [interact_rows[:h], upb, interact_rows[h:], upb]).reshape(-1, G)

    parts_lo = _seg_call(WLO)(
        gidx_b, sidx_b, node_lo, jnp.zeros((ACC_B, WLO), f32))
    parts_hi = _seg_call(WHI)(
        gidx_b, sidx_b, node_hi, jnp.zeros((ACC_B, WHI), f32))

    bl = 1000
    node_res = pl.pallas_call(
        _norm_body,
        out_shape=jax.ShapeDtypeStruct((N_NODE, D), f32),
        grid=(N_NODE // bl,),
        in_specs=[pl.BlockSpec((bl, WLO), lambda i: (i, 0)),
                  pl.BlockSpec((bl, WHI), lambda i: (i, 0)),
                  pl.BlockSpec((bl, D), lambda i: (i, 0))],
        out_specs=pl.BlockSpec((bl, D), lambda i: (i, 0)),
    )(node_lo, node_hi, all_embedding)

    user_res = pl.pallas_call(
        _user_body,
        out_shape=jax.ShapeDtypeStruct((N_USERS, D), f32),
        grid=(N_USERS // bl,),
        in_specs=[pl.BlockSpec((bl, D), lambda i: (i, 0)),
                  pl.BlockSpec((NC, bl, WLO), lambda i: (0, i, 0)),
                  pl.BlockSpec((NC, bl, WHI), lambda i: (0, i, 0))],
        out_specs=pl.BlockSpec((bl, D), lambda i: (i, 0)),
    )(user_embedding, parts_lo, parts_hi)

    return (user_res, node_res)


# speedup vs baseline: 2.1077x; 1.0668x over previous
import jax
import jax.numpy as jnp
from jax import lax
from jax.experimental import pallas as pl
from jax.experimental.pallas import tpu as pltpu
from jax.experimental.pallas import tpu_sc as plsc

N_USERS = 20000
N_NEWS = 10000
N_ENT = 25000
N_NODE = N_NEWS + N_ENT
D = 100
DP = 104
WLO = 56
WHI = 48
NEIGH = 20
NNZ = 500000

NC = 2
NS = 16
NW = NC * NS
OFFS = (0, 16, 32, 48, 64, 80, 88)

RPG = 12
TPR0 = 1464
TPR1 = 744
GPA0 = TPR0 // RPG
GPA1 = TPR1 // RPG
N_PAD = NS * (TPR0 + TPR1)

G = 256
ACC_B = 20096
SB = ACC_B // NS
GPB = 62
CHB = (32, 30)
EPS_B = NS * GPB * G
PAD_B = EPS_B - NNZ // 2


def _agg_body(gidx_hbm, base_hbm, table_hbm, lo_hbm, hi_hbm,
              islot, gb0, gb1, bb0, bb1, ob0, ob1,
              sg0, sg1, sb0, sb1, sl0, sl1, sh0, sh1):
    gbufs, bbufs, obufs = (gb0, gb1), (bb0, bb1), (ob0, ob1)
    sgs, sbs, sls, shs = (sg0, sg1), (sb0, sb1), (sl0, sl1), (sh0, sh1)
    c = lax.axis_index("c")
    t = lax.axis_index("s")
    row_t = jnp.where(c == 0, t * TPR0, NS * TPR0 + t * TPR1)
    ng = jnp.where(c == 0, GPA0, GPA1)
    pltpu.sync_copy(gidx_hbm.at[pl.ds(row_t // RPG, GPA0)], islot)
    for s in range(2):
        pltpu.async_copy(table_hbm.at[islot.at[s]], gbufs[s], sgs[s])
        pltpu.async_copy(base_hbm.at[pl.ds(row_t + s * RPG, RPG)],
                         bbufs[s], sbs[s])

    def body(i, carry):
        for s in range(2):
            g = 2 * i + s
            rows = row_t + g * RPG
            pltpu.make_async_copy(
                table_hbm.at[islot.at[g]], gbufs[s], sgs[s]).wait()
            pltpu.make_async_copy(
                base_hbm.at[pl.ds(rows, RPG)], bbufs[s], sbs[s]).wait()

            @pl.when(i > 0)
            def _():
                pltpu.make_async_copy(
                    obufs[s].at[:, pl.ds(0, WLO)],
                    lo_hbm.at[pl.ds(rows, RPG)], sls[s]).wait()
                pltpu.make_async_copy(
                    obufs[s].at[:, pl.ds(WLO, WHI)],
                    hi_hbm.at[pl.ds(rows, RPG)], shs[s]).wait()

            def red(r, c):
                for off in OFFS:
                    acc = bbufs[s][r, pl.ds(off, 16)]
                    for j in range(NEIGH):
                        acc = acc + gbufs[s][r * NEIGH + j, pl.ds(off, 16)]
                    obufs[s][r, pl.ds(off, 16)] = acc
                return c

            lax.fori_loop(0, RPG, red, 0)
            pltpu.async_copy(obufs[s].at[:, pl.ds(0, WLO)],
                             lo_hbm.at[pl.ds(rows, RPG)], sls[s])
            pltpu.async_copy(obufs[s].at[:, pl.ds(WLO, WHI)],
                             hi_hbm.at[pl.ds(rows, RPG)], shs[s])

            @pl.when(i < ng // 2 - 1)
            def _():
                pltpu.async_copy(table_hbm.at[islot.at[g + 2]],
                                 gbufs[s], sgs[s])
                pltpu.async_copy(base_hbm.at[pl.ds(rows + 2 * RPG, RPG)],
                                 bbufs[s], sbs[s])
        return carry

    lax.fori_loop(0, ng // 2, body, 0)
    for s in range(2):
        rows = row_t + (ng - 2 + s) * RPG
        pltpu.make_async_copy(obufs[s].at[:, pl.ds(0, WLO)],
                              lo_hbm.at[pl.ds(rows, RPG)], sls[s]).wait()
        pltpu.make_async_copy(obufs[s].at[:, pl.ds(WLO, WHI)],
                              hi_hbm.at[pl.ds(rows, RPG)], shs[s]).wait()


def _agg_call():
    return pl.kernel(
        _agg_body,
        out_type=(jax.ShapeDtypeStruct((N_PAD, WLO), jnp.float32),
                  jax.ShapeDtypeStruct((N_PAD, WHI), jnp.float32)),
        mesh=plsc.VectorSubcoreMesh(core_axis_name="c", subcore_axis_name="s",
                                    num_cores=NC, num_subcores=NS),
        scratch_types=(
            [pltpu.VMEM((GPA0, RPG * NEIGH), jnp.int32)]
            + [pltpu.VMEM((RPG * NEIGH, DP), jnp.float32)] * 2
            + [pltpu.VMEM((RPG, DP), jnp.float32)] * 4
            + [pltpu.SemaphoreType.DMA] * 8
        ),
        compiler_params=pltpu.CompilerParams(use_tc_tiling_on_sc=False),
    )


def _pipe_loop(table_hbm, acc, gslot, sslot, bufs, sgs, sss, n_groups):
    pltpu.async_copy(table_hbm.at[gslot.at[0]], bufs[0], sgs[0])

    def body(i, carry):
        g0 = 2 * i
        pltpu.make_async_copy(table_hbm.at[gslot.at[g0]], bufs[0], sgs[0]).wait()

        @pl.when(g0 > 0)
        def _():
            pltpu.make_async_copy(bufs[1], acc.at[sslot.at[0]], sss[1]).wait()

        pltpu.async_copy(table_hbm.at[gslot.at[g0 + 1]], bufs[1], sgs[1])
        pltpu.async_copy(bufs[0], acc.at[sslot.at[g0]], sss[0], add=True)
        pltpu.make_async_copy(
            table_hbm.at[gslot.at[g0 + 1]], bufs[1], sgs[1]).wait()
        pltpu.make_async_copy(bufs[0], acc.at[sslot.at[0]], sss[0]).wait()

        @pl.when(g0 + 2 < n_groups)
        def _():
            pltpu.async_copy(table_hbm.at[gslot.at[g0 + 2]], bufs[0], sgs[0])

        pltpu.async_copy(bufs[1], acc.at[sslot.at[g0 + 1]], sss[1], add=True)
        return carry

    lax.fori_loop(0, n_groups // 2, body, 0)
    pltpu.make_async_copy(bufs[1], acc.at[sslot.at[0]], sss[1]).wait()


def _seg_body(gidx_hbm, sidx_hbm, table_hbm, init_hbm, out_hbm,
              acc, gslot, sslot, buf0, buf1, sg0, sg1, ss0, ss1):
    sc = lax.axis_index("c")
    t = lax.axis_index("s")
    pltpu.sync_copy(init_hbm.at[pl.ds(t * SB, SB)], acc.at[pl.ds(t * SB, SB)])
    plsc.subcore_barrier()
    base = (sc * NS + t) * GPB
    off = 0
    for ch in CHB:
        pltpu.sync_copy(gidx_hbm.at[pl.ds(base + off, ch)],
                        gslot.at[pl.ds(0, ch)])
        pltpu.sync_copy(sidx_hbm.at[pl.ds(base + off, ch)],
                        sslot.at[pl.ds(0, ch)])
        _pipe_loop(table_hbm, acc, gslot, sslot, (buf0, buf1),
                   (sg0, sg1), (ss0, ss1), ch)
        off += ch
    plsc.subcore_barrier()
    pltpu.sync_copy(acc.at[pl.ds(t * SB, SB)],
                    out_hbm.at[sc, pl.ds(t * SB, SB)])


def _seg_call(width):
    return pl.kernel(
        _seg_body,
        out_type=jax.ShapeDtypeStruct((NC, ACC_B, width), jnp.float32),
        mesh=plsc.VectorSubcoreMesh(core_axis_name="c", subcore_axis_name="s",
                                    num_cores=NC, num_subcores=NS),
        scratch_types=[
            pltpu.VMEM_SHARED((ACC_B, width), jnp.float32),
            pltpu.VMEM((max(CHB), G), jnp.int32),
            pltpu.VMEM((max(CHB), G), jnp.int32),
            pltpu.VMEM((G, width), jnp.float32),
            pltpu.VMEM((G, width), jnp.float32),
            pltpu.SemaphoreType.DMA,
            pltpu.SemaphoreType.DMA,
            pltpu.SemaphoreType.DMA,
            pltpu.SemaphoreType.DMA,
        ],
        compiler_params=pltpu.CompilerParams(use_tc_tiling_on_sc=False),
    )


def _norm_body(lo_ref, hi_ref, a_ref, o_ref):
    x = jnp.concatenate([lo_ref[...], hi_ref[...][:, :D - WLO]], axis=1)
    n = jnp.maximum(jnp.sqrt(jnp.sum(x * x, axis=1, keepdims=True)), 1e-12)
    o_ref[...] = a_ref[...] + 2.0 * (x / n)


def _user_body(u_ref, plo_ref, phi_ref, o_ref):
    u = u_ref[...]
    agg = jnp.concatenate(
        [plo_ref[0] + plo_ref[1], (phi_ref[0] + phi_ref[1])[:, :D - WLO]],
        axis=1)
    x = u + agg
    n = jnp.maximum(jnp.sqrt(jnp.sum(x * x, axis=1, keepdims=True)), 1e-12)
    o_ref[...] = u + 2.0 * (x / n)


def kernel(user_embedding, all_embedding, entity_embedding, relation_embedding,
           news_entities, neigh_entities, neigh_relations,
           interact_rows, interact_cols, interact_vals,
           W_news, b_news, W_ent, b_ent):
    f32, i32 = jnp.float32, jnp.int32

    ent_pad = jnp.pad(entity_embedding, ((0, 0), (0, DP - D)))
    gidx_a = jnp.concatenate(
        [news_entities.reshape(-1), neigh_entities.reshape(-1),
         jnp.zeros(((N_PAD - N_NODE) * NEIGH
                    + (GPA0 - GPA1) * RPG * NEIGH,), i32)]
    ).reshape(-1, RPG * NEIGH)
    base_pad = jnp.pad(
        jnp.concatenate([all_embedding[:N_NEWS], all_embedding[:N_ENT]],
                        axis=0),
        ((0, N_PAD - N_NODE), (0, DP - D)))

    node_lo, node_hi = _agg_call()(gidx_a, base_pad, ent_pad)

    h = NNZ // 2
    zpb = jnp.zeros((PAD_B,), i32)
    upb = jnp.full((PAD_B,), N_USERS, i32)
    gidx_b = jnp.concatenate(
        [interact_cols[:h], zpb, interact_cols[h:], zpb]).reshape(-1, G)
    sidx_b = jnp.concatenate(
        [interact_rows[:h], upb, interact_rows[h:], upb]).reshape(-1, G)

    parts_lo = _seg_call(WLO)(
        gidx_b, sidx_b, node_lo, jnp.zeros((ACC_B, WLO), f32))
    parts_hi = _seg_call(WHI)(
        gidx_b, sidx_b, node_hi, jnp.zeros((ACC_B, WHI), f32))

    bl = 1000
    node_res = pl.pallas_call(
        _norm_body,
        out_shape=jax.ShapeDtypeStruct((N_NODE, D), f32),
        grid=(N_NODE // bl,),
        in_specs=[pl.BlockSpec((bl, WLO), lambda i: (i, 0)),
                  pl.BlockSpec((bl, WHI), lambda i: (i, 0)),
                  pl.BlockSpec((bl, D), lambda i: (i, 0))],
        out_specs=pl.BlockSpec((bl, D), lambda i: (i, 0)),
    )(node_lo, node_hi, all_embedding)

    user_res = pl.pallas_call(
        _user_body,
        out_shape=jax.ShapeDtypeStruct((N_USERS, D), f32),
        grid=(N_USERS // bl,),
        in_specs=[pl.BlockSpec((bl, D), lambda i: (i, 0)),
                  pl.BlockSpec((NC, bl, WLO), lambda i: (0, i, 0)),
                  pl.BlockSpec((NC, bl, WHI), lambda i: (0, i, 0))],
        out_specs=pl.BlockSpec((bl, D), lambda i: (i, 0)),
    )(user_embedding, parts_lo, parts_hi)

    return (user_res, node_res)
